# baseline (speedup 1.0000x reference)
import functools

import jax
import jax.numpy as jnp
import numpy as np
from jax import lax
from jax.experimental import pallas as pl
from jax.experimental.pallas import tpu as pltpu
from jax.experimental.pallas import tpu_sc as plsc

_N = 1_000_000
_D = 16
_NZ = 500_000
_NC = 2
_NS = 16
_NW = _NC * _NS
_CCH = 320
_NCHUNK = _N // _CCH
_RING = 2
_GROUPS = 98


@functools.cache
def _build_index_table():
    idx = np.sort(np.asarray(jax.random.permutation(jax.random.key(42), _N)[:_NZ]))
    chunk_of = idx // _CCH
    local = (idx % _CCH).astype(np.int32)
    counts = np.bincount(chunk_of, minlength=_NCHUNK)
    assert counts.min() > 0
    k = int(-(-counts.max() // 16) * 16)
    tab = np.empty((_NCHUNK, k), np.int32)
    off = 0
    for ch in range(_NCHUNK):
        c = counts[ch]
        tab[ch, :c] = local[off:off + c]
        tab[ch, c:] = local[off]
        off += c
    return tab


try:
    _K_CH = _build_index_table().shape[1]
except Exception:
    _K_CH = None


@functools.cache
def _get_sc_kernel():
    k_ch = _build_index_table().shape[1]
    mesh = plsc.VectorSubcoreMesh(
        core_axis_name="c", subcore_axis_name="s", num_cores=_NC, num_subcores=_NS
    )

    @functools.partial(
        pl.kernel,
        out_type=(
            jax.ShapeDtypeStruct((_N, _D), jnp.float32),
            jax.ShapeDtypeStruct((_N, _D), jnp.float32),
        ),
        mesh=mesh,
        compiler_params=pltpu.CompilerParams(needs_layout_passes=False),
        scratch_types=(
            [pltpu.VMEM((_CCH, _D), jnp.float32) for _ in range(_RING)]
            + [pltpu.VMEM((k_ch,), jnp.int32) for _ in range(_RING)]
            + [pltpu.SemaphoreType.DMA for _ in range(2 * _RING)]
        ),
    )
    def _sc_dropout(emb1, emb2, idx_hbm, out1, out2, *scratch):
        bufs = scratch[:_RING]
        idxs = scratch[_RING:2 * _RING]
        insems = scratch[2 * _RING:3 * _RING]
        outsems = scratch[3 * _RING:4 * _RING]

        c = lax.axis_index("c")
        s = lax.axis_index("s")
        wid = s * _NC + c
        zvec = jnp.zeros((_D,), jnp.float32)

        def chunk_id(g):
            return g * _NW + wid

        def fire_in(src, g, b):
            ch = chunk_id(g)

            @pl.when(ch < _NCHUNK)
            def _():
                pltpu.async_copy(src.at[pl.ds(ch * _CCH, _CCH)], bufs[b], insems[b])
                pltpu.async_copy(idx_hbm.at[ch], idxs[b], insems[b])

        def process(src, dst, g, b):
            ch = chunk_id(g)

            @pl.when(ch < _NCHUNK)
            def _():
                pltpu.make_async_copy(
                    src.at[pl.ds(ch * _CCH, _CCH)], bufs[b], insems[b]
                ).wait()
                pltpu.make_async_copy(idx_hbm.at[ch], idxs[b], insems[b]).wait()

                @pl.loop(0, k_ch // 16)
                def _zero(grp):
                    rvec = idxs[b][pl.ds(grp * 16, 16)]
                    for col in range(_D):
                        plsc.store_scatter(
                            bufs[b],
                            [rvec, jnp.full((16,), col, jnp.int32)],
                            zvec,
                        )

                pltpu.async_copy(bufs[b], dst.at[pl.ds(ch * _CCH, _CCH)], outsems[b])
                pltpu.make_async_copy(
                    bufs[b], dst.at[pl.ds(ch * _CCH, _CCH)], outsems[b]
                ).wait()

        srcs = (emb1, emb2)
        dsts = (out1, out2)
        for b in range(_RING):
            fire_in(srcs[b % 2], b // 2, b)

        @pl.loop(0, _GROUPS)
        def _run(jg):
            for b in range(_RING):
                t = b % 2
                g = jg
                process(srcs[t], dsts[t], g, b)

                @pl.when(jg < _GROUPS - 1)
                def _prefetch():
                    fire_in(srcs[t], g + 1, b)

    return _sc_dropout


def kernel(emb1, emb2):
    idx_tab = jnp.asarray(_build_index_table())
    return _get_sc_kernel()(emb1, emb2, idx_tab)
